# Initial kernel scaffold; baseline (speedup 1.0000x reference)
import jax
import jax.numpy as jnp
from jax.experimental import pallas as pl


def kernel(codon_indices, usage_freqs, tai_weights):
    raise NotImplementedError("write your pallas kernel here")


# speedup vs baseline: 5.7123x; 5.7123x over previous
import functools

import jax
import jax.numpy as jnp
from jax import lax
from jax.experimental import pallas as pl
from jax.experimental.pallas import tpu as pltpu
from jax.experimental.pallas import tpu_sc as plsc

L = 16

TAI_BITS = 15
TAI_SCALE = (1 << TAI_BITS) - 1
LOG_BITS = 16
LOG_SCALE = (1 << LOG_BITS) - 1


def _make_sc_call(B, W, num_workers):
  rows_per = B // num_workers
  ch = L
  nchunks = rows_per // ch
  unroll = 8
  mesh = plsc.VectorSubcoreMesh(core_axis_name="c", subcore_axis_name="s")
  info = plsc.get_sparse_core_info()
  nc = info.num_cores

  @functools.partial(
      pl.kernel,
      mesh=mesh,
      compiler_params=pltpu.CompilerParams(needs_layout_passes=False),
      out_type=[
          jax.ShapeDtypeStruct((B,), jnp.float32),
          jax.ShapeDtypeStruct((B,), jnp.float32),
      ],
      scratch_types=[
          pltpu.VMEM((64,), jnp.int32),
          pltpu.VMEM((L,), jnp.float32),
          pltpu.VMEM((ch * W,), jnp.int32),
          pltpu.VMEM((ch * W,), jnp.int32),
          pltpu.VMEM((rows_per,), jnp.float32),
          pltpu.VMEM((rows_per,), jnp.float32),
          pltpu.SemaphoreType.DMA,
          pltpu.SemaphoreType.DMA,
      ],
  )
  def sc_kernel(idx_hbm, lut_hbm, par_hbm, tai_hbm, cai_hbm,
                lut_v, par_v, buf0, buf1, ot_v, oc_v, sem0, sem1):
    wid = lax.axis_index("s") * nc + lax.axis_index("c")
    base = wid * rows_per
    pltpu.sync_copy(lut_hbm, lut_v)
    pltpu.sync_copy(par_hbm, par_v)
    bufs = (buf0, buf1)
    sems = (sem0, sem1)

    pv = par_v[pl.ds(0, L)]
    c_tai = pv[0]
    c_log = pv[1]
    row_pos0 = lax.iota(jnp.int32, L) * W

    def chunk_src(c):
      return idx_hbm.at[pl.ds((base + c * ch) * W, ch * W)]

    pltpu.async_copy(chunk_src(0), bufs[0], sems[0])

    def pair_body(c2, _):
      for slot in range(2):
        c = c2 * 2 + slot
        pltpu.make_async_copy(chunk_src(c), bufs[slot], sems[slot]).wait()

        @pl.when(c + 1 < nchunks)
        def _(c=c, slot=slot):
          pltpu.async_copy(chunk_src(c + 1), bufs[1 - slot], sems[1 - slot])

        def vec_body(j, acc, buf=bufs[slot]):
          acc_t, acc_q, pos = acc
          for _ in range(unroll):
            v = plsc.load_gather(buf, [pos])
            pos = pos + 1
            p = plsc.load_gather(lut_v, [v])
            acc_t = acc_t + lax.shift_right_logical(p, LOG_BITS)
            acc_q = acc_q + lax.bitwise_and(p, jnp.int32(LOG_SCALE))
          return (acc_t, acc_q, pos)

        zero = jnp.zeros((L,), jnp.int32)
        acc_t, acc_q, _ = lax.fori_loop(0, W // unroll, vec_body,
                                        (zero, zero, row_pos0))
        ot_v[pl.ds(c * ch, L)] = acc_t.astype(jnp.float32) * c_tai
        oc_v[pl.ds(c * ch, L)] = jnp.exp(acc_q.astype(jnp.float32) * c_log)
      return 0

    lax.fori_loop(0, nchunks // 2, pair_body, 0)
    pltpu.sync_copy(ot_v, tai_hbm.at[pl.ds(base, rows_per)])
    pltpu.sync_copy(oc_v, cai_hbm.at[pl.ds(base, rows_per)])

  return sc_kernel


def kernel(codon_indices, usage_freqs, tai_weights):
  B, W = codon_indices.shape
  info = plsc.get_sparse_core_info()
  num_workers = info.num_cores * info.num_subcores

  neg_log = -jnp.log(usage_freqs / jnp.max(usage_freqs) + 1e-8)
  neg_log = jnp.maximum(neg_log, 0.0)
  qscale = LOG_SCALE / jnp.maximum(jnp.max(neg_log), 1e-30)
  qlog = jnp.clip(jnp.round(neg_log * qscale), 0, LOG_SCALE).astype(jnp.int32)
  tmax = jnp.maximum(jnp.max(tai_weights), 1e-30)
  qtai = jnp.clip(jnp.round(tai_weights * (TAI_SCALE / tmax)), 0,
                  TAI_SCALE).astype(jnp.int32)
  packed = jnp.bitwise_or(jnp.left_shift(qtai, LOG_BITS), qlog)

  c_tai = tmax / (TAI_SCALE * float(W))
  c_log = -1.0 / (qscale * float(W))
  params = jnp.zeros((L,), jnp.float32).at[0].set(c_tai).at[1].set(c_log)

  sc_call = _make_sc_call(B, W, num_workers)
  tai, cai = sc_call(codon_indices.reshape(-1), packed, params)
  return tai, cai


# speedup vs baseline: 6.4462x; 1.1285x over previous
import functools

import jax
import jax.numpy as jnp
from jax import lax
from jax.experimental import pallas as pl
from jax.experimental.pallas import tpu as pltpu
from jax.experimental.pallas import tpu_sc as plsc

L = 16

QBITS = 12
QMAX = (1 << QBITS) - 1
TAI_SHIFT = QBITS + 3
FLUSH = 8
LOW_MASK = (1 << TAI_SHIFT) - 1


def _make_sc_call(B, W, num_workers):
  rows_per = B // num_workers
  ch = L
  nchunks = rows_per // ch
  ws = W + 1
  mesh = plsc.VectorSubcoreMesh(core_axis_name="c", subcore_axis_name="s")
  info = plsc.get_sparse_core_info()
  nc = info.num_cores

  @functools.partial(
      pl.kernel,
      mesh=mesh,
      compiler_params=pltpu.CompilerParams(needs_layout_passes=False),
      out_type=[
          jax.ShapeDtypeStruct((B,), jnp.float32),
          jax.ShapeDtypeStruct((B,), jnp.float32),
      ],
      scratch_types=[
          pltpu.VMEM((64 * L,), jnp.int32),
          pltpu.VMEM((L,), jnp.float32),
          pltpu.VMEM((ch, ws), jnp.int32),
          pltpu.VMEM((ch, ws), jnp.int32),
          pltpu.VMEM((rows_per,), jnp.float32),
          pltpu.VMEM((rows_per,), jnp.float32),
          pltpu.SemaphoreType.DMA,
          pltpu.SemaphoreType.DMA,
      ],
  )
  def sc_kernel(idx_hbm, lut_hbm, par_hbm, tai_hbm, cai_hbm,
                lut_v, par_v, buf0, buf1, ot_v, oc_v, sem0, sem1):
    wid = lax.axis_index("s") * nc + lax.axis_index("c")
    base = wid * rows_per
    pltpu.sync_copy(lut_hbm, lut_v)
    pltpu.sync_copy(par_hbm, par_v)
    bufs = (buf0, buf1)
    sems = (sem0, sem1)

    pv = par_v[pl.ds(0, L)]
    c_tai = pv[0]
    c_log = pv[1]
    lane = lax.iota(jnp.int32, L)
    row_iota = lane

    def start_chunk(c, slot):
      for r in range(ch):
        pltpu.async_copy(idx_hbm.at[base + c * ch + r],
                         bufs[slot].at[r, pl.ds(0, W)], sems[slot])

    def wait_chunk(c, slot):
      for r in range(ch):
        pltpu.make_async_copy(idx_hbm.at[base + c * ch + r],
                              bufs[slot].at[r, pl.ds(0, W)],
                              sems[slot]).wait()

    start_chunk(0, 0)

    def pair_body(c2, _):
      for slot in range(2):
        c = c2 * 2 + slot
        wait_chunk(c, slot)

        @pl.when(c + 1 < nchunks)
        def _(c=c, slot=slot):
          start_chunk(c + 1, 1 - slot)

        def vec_body(j, acc, buf=bufs[slot]):
          acc_t, acc_q = acc
          col0 = j * FLUSH
          raw = jnp.zeros((L,), jnp.int32)
          for u in range(FLUSH):
            colv = jnp.full((L,), col0, jnp.int32) | u
            v = plsc.load_gather(buf, [row_iota, colv])
            a = lax.shift_left(v, 4) | lane
            raw = raw + plsc.load_gather(lut_v, [a])
          acc_t = acc_t + lax.shift_right_logical(raw, TAI_SHIFT)
          acc_q = acc_q + lax.bitwise_and(raw, jnp.int32(LOW_MASK))
          return (acc_t, acc_q)

        zero = jnp.zeros((L,), jnp.int32)
        acc_t, acc_q = lax.fori_loop(0, W // FLUSH, vec_body, (zero, zero))
        ot_v[pl.ds(c * ch, L)] = acc_t.astype(jnp.float32) * c_tai
        oc_v[pl.ds(c * ch, L)] = jnp.exp(acc_q.astype(jnp.float32) * c_log)
      return 0

    lax.fori_loop(0, nchunks // 2, pair_body, 0)
    pltpu.sync_copy(ot_v, tai_hbm.at[pl.ds(base, rows_per)])
    pltpu.sync_copy(oc_v, cai_hbm.at[pl.ds(base, rows_per)])

  return sc_kernel


def kernel(codon_indices, usage_freqs, tai_weights):
  B, W = codon_indices.shape
  info = plsc.get_sparse_core_info()
  num_workers = info.num_cores * info.num_subcores

  neg_log = -jnp.log(usage_freqs / jnp.max(usage_freqs) + 1e-8)
  neg_log = jnp.maximum(neg_log, 0.0)
  qscale = QMAX / jnp.maximum(jnp.max(neg_log), 1e-30)
  qlog = jnp.clip(jnp.round(neg_log * qscale), 0, QMAX).astype(jnp.int32)
  tmax = jnp.maximum(jnp.max(tai_weights), 1e-30)
  qtai = jnp.clip(jnp.round(tai_weights * (QMAX / tmax)), 0,
                  QMAX).astype(jnp.int32)
  packed = jnp.bitwise_or(jnp.left_shift(qtai, TAI_SHIFT), qlog)
  packed_rep = jnp.repeat(packed, L)

  c_tai = tmax / (QMAX * float(W))
  c_log = -1.0 / (qscale * float(W))
  params = jnp.zeros((L,), jnp.float32).at[0].set(c_tai).at[1].set(c_log)

  sc_call = _make_sc_call(B, W, num_workers)
  tai, cai = sc_call(codon_indices, packed_rep, params)
  return tai, cai
